# Initial kernel scaffold; baseline (speedup 1.0000x reference)
import functools
import jax
import jax.numpy as jnp
from jax import lax
from jax.experimental import pallas as pl
from jax.experimental.pallas import tpu as pltpu
from jax.experimental.pallas import tpu_sc as plsc

VOCAB = 1000000
EMBED_DIM = 64
BATCH = 16384
HIST = 50

NC = 2
NS = 16
NW = NC * NS

B = BATCH * HIST
B_PER_W = B // NW
CH = 128
CHUNKS = B_PER_W // CH

_mesh = plsc.VectorSubcoreMesh(
    core_axis_name="c", subcore_axis_name="s", num_cores=NC, num_subcores=NS
)


@functools.partial(
    pl.kernel,
    out_type=jax.ShapeDtypeStruct((B, EMBED_DIM), jnp.float32),
    mesh=_mesh,
    scratch_types=[
        pltpu.VMEM((CHUNKS, CH), jnp.int32),
        pltpu.VMEM((CH, EMBED_DIM), jnp.float32),
        pltpu.SemaphoreType.DMA,
    ],
)
def _gather_kernel(idx_hbm, table_hbm, out_hbm, idx_v, rows_v, gsem):
    wid = lax.axis_index("s") * NC + lax.axis_index("c")
    base = wid * B_PER_W
    pltpu.sync_copy(idx_hbm.at[wid], idx_v)

    def chunk_body(i, carry):
        pltpu.async_copy(table_hbm.at[idx_v.at[i]], rows_v, gsem).wait()
        pltpu.sync_copy(rows_v, out_hbm.at[pl.ds(base + i * CH, CH)])
        return carry

    lax.fori_loop(0, CHUNKS, chunk_body, 0)


def kernel(X, table):
    idx = X.reshape(NW, CHUNKS, CH).astype(jnp.int32)
    out = _gather_kernel(idx, table)
    return out.reshape(BATCH, HIST, EMBED_DIM)


# speedup vs baseline: 1.6856x; 1.6856x over previous
import functools
import jax
import jax.numpy as jnp
from jax import lax
from jax.experimental import pallas as pl
from jax.experimental.pallas import tpu as pltpu
from jax.experimental.pallas import tpu_sc as plsc

VOCAB = 1000000
EMBED_DIM = 64
BATCH = 16384
HIST = 50

NC = 2
NS = 16
NW = NC * NS

B = BATCH * HIST
B_PER_W = B // NW
CH = 128
CHUNKS = B_PER_W // CH

_mesh = plsc.VectorSubcoreMesh(
    core_axis_name="c", subcore_axis_name="s", num_cores=NC, num_subcores=NS
)


@functools.partial(
    pl.kernel,
    out_type=jax.ShapeDtypeStruct((B, EMBED_DIM), jnp.float32),
    mesh=_mesh,
    scratch_types=[
        pltpu.VMEM((CHUNKS, CH), jnp.int32),
        pltpu.VMEM((CH, EMBED_DIM), jnp.float32),
        pltpu.SemaphoreType.DMA,
    ],
    compiler_params=pltpu.CompilerParams(use_tc_tiling_on_sc=False),
)
def _gather_kernel(idx_hbm, table_hbm, out_hbm, idx_v, rows_v, gsem):
    wid = lax.axis_index("s") * NC + lax.axis_index("c")
    base = wid * B_PER_W
    pltpu.sync_copy(idx_hbm.at[wid], idx_v)

    def chunk_body(i, carry):
        pltpu.async_copy(table_hbm.at[idx_v.at[i]], rows_v, gsem).wait()
        pltpu.sync_copy(rows_v, out_hbm.at[pl.ds(base + i * CH, CH)])
        return carry

    lax.fori_loop(0, CHUNKS, chunk_body, 0)


def kernel(X, table):
    idx = X.reshape(NW, CHUNKS, CH).astype(jnp.int32)
    out = _gather_kernel(idx, table)
    return out.reshape(BATCH, HIST, EMBED_DIM)


# speedup vs baseline: 1.8750x; 1.1123x over previous
import functools
import jax
import jax.numpy as jnp
from jax import lax
from jax.experimental import pallas as pl
from jax.experimental.pallas import tpu as pltpu
from jax.experimental.pallas import tpu_sc as plsc

VOCAB = 1000000
EMBED_DIM = 64
BATCH = 16384
HIST = 50

NC = 2
NS = 16
NW = NC * NS

B = BATCH * HIST
B_PER_W = B // NW
CH = 128
CHUNKS = B_PER_W // CH
NBUF = 8
LOOK = 4
NITER = CHUNKS // NBUF

_mesh = plsc.VectorSubcoreMesh(
    core_axis_name="c", subcore_axis_name="s", num_cores=NC, num_subcores=NS
)


@functools.partial(
    pl.kernel,
    out_type=jax.ShapeDtypeStruct((B, EMBED_DIM), jnp.float32),
    mesh=_mesh,
    scratch_types=[
        pltpu.VMEM((CHUNKS, CH), jnp.int32),
        pltpu.VMEM((NBUF, CH, EMBED_DIM), jnp.float32),
        [pltpu.SemaphoreType.DMA] * NBUF,
        [pltpu.SemaphoreType.DMA] * NBUF,
    ],
    compiler_params=pltpu.CompilerParams(use_tc_tiling_on_sc=False),
)
def _gather_kernel(idx_hbm, table_hbm, out_hbm, idx_v, rows_v, gsems, osems):
    wid = lax.axis_index("s") * NC + lax.axis_index("c")
    base = wid * B_PER_W
    pltpu.sync_copy(idx_hbm.at[wid], idx_v)

    def fire_gather(chunk, slot):
        pltpu.async_copy(table_hbm.at[idx_v.at[chunk]], rows_v.at[slot], gsems[slot])

    def out_ref_for(chunk):
        return out_hbm.at[pl.ds(base + chunk * CH, CH)]

    for b in range(LOOK):
        fire_gather(b, b)

    def titer(t, carry):
        for b in range(NBUF):
            i = t * NBUF + b
            j = i + LOOK
            sj = (b + LOOK) % NBUF

            @pl.when(j < CHUNKS)
            def _():
                @pl.when(j >= NBUF)
                def _():
                    pltpu.make_async_copy(
                        rows_v.at[sj], out_ref_for(j - NBUF), osems[sj]
                    ).wait()
                fire_gather(j, sj)

            pltpu.make_async_copy(
                table_hbm.at[idx_v.at[i]], rows_v.at[b], gsems[b]
            ).wait()
            pltpu.async_copy(rows_v.at[b], out_ref_for(i), osems[b])
        return carry

    lax.fori_loop(0, NITER, titer, 0)

    for b in range(NBUF):
        last = CHUNKS - NBUF + b
        pltpu.make_async_copy(rows_v.at[b], out_ref_for(last), osems[b]).wait()


def kernel(X, table):
    idx = X.reshape(NW, CHUNKS, CH).astype(jnp.int32)
    out = _gather_kernel(idx, table)
    return out.reshape(BATCH, HIST, EMBED_DIM)


# speedup vs baseline: 1.8760x; 1.0005x over previous
import functools
import jax
import jax.numpy as jnp
from jax import lax
from jax.experimental import pallas as pl
from jax.experimental.pallas import tpu as pltpu
from jax.experimental.pallas import tpu_sc as plsc

VOCAB = 1000000
EMBED_DIM = 64
BATCH = 16384
HIST = 50

NC = 2
NS = 16
NW = NC * NS

B = BATCH * HIST
B_PER_W = B // NW
CH = 256
CHUNKS = B_PER_W // CH
NBUF = 5
LOOK = 3
NITER = CHUNKS // NBUF

_mesh = plsc.VectorSubcoreMesh(
    core_axis_name="c", subcore_axis_name="s", num_cores=NC, num_subcores=NS
)


@functools.partial(
    pl.kernel,
    out_type=jax.ShapeDtypeStruct((B, EMBED_DIM), jnp.float32),
    mesh=_mesh,
    scratch_types=[
        pltpu.VMEM((CHUNKS, CH), jnp.int32),
        pltpu.VMEM((NBUF, CH, EMBED_DIM), jnp.float32),
        [pltpu.SemaphoreType.DMA] * NBUF,
        [pltpu.SemaphoreType.DMA] * NBUF,
    ],
    compiler_params=pltpu.CompilerParams(use_tc_tiling_on_sc=False),
)
def _gather_kernel(idx_hbm, table_hbm, out_hbm, idx_v, rows_v, gsems, osems):
    wid = lax.axis_index("s") * NC + lax.axis_index("c")
    base = wid * B_PER_W
    pltpu.sync_copy(idx_hbm.at[wid], idx_v)

    def fire_gather(chunk, slot):
        pltpu.async_copy(table_hbm.at[idx_v.at[chunk]], rows_v.at[slot], gsems[slot])

    def out_ref_for(chunk):
        return out_hbm.at[pl.ds(base + chunk * CH, CH)]

    for b in range(LOOK):
        fire_gather(b, b)

    def titer(t, carry):
        for b in range(NBUF):
            i = t * NBUF + b
            j = i + LOOK
            sj = (b + LOOK) % NBUF

            @pl.when(j < CHUNKS)
            def _():
                @pl.when(j >= NBUF)
                def _():
                    pltpu.make_async_copy(
                        rows_v.at[sj], out_ref_for(j - NBUF), osems[sj]
                    ).wait()
                fire_gather(j, sj)

            pltpu.make_async_copy(
                table_hbm.at[idx_v.at[i]], rows_v.at[b], gsems[b]
            ).wait()
            pltpu.async_copy(rows_v.at[b], out_ref_for(i), osems[b])
        return carry

    lax.fori_loop(0, NITER, titer, 0)

    for b in range(NBUF):
        last = CHUNKS - NBUF + b
        pltpu.make_async_copy(rows_v.at[b], out_ref_for(last), osems[b]).wait()


def kernel(X, table):
    idx = X.reshape(NW, CHUNKS, CH).astype(jnp.int32)
    out = _gather_kernel(idx, table)
    return out.reshape(BATCH, HIST, EMBED_DIM)
